# baseline (speedup 1.0000x reference)
import functools

import jax
import jax.numpy as jnp
from jax import lax
from jax.experimental import pallas as pl
from jax.experimental.pallas import tpu as pltpu
from jax.experimental.pallas import tpu_sc as plsc

_NC = 2
_NS = 16
_L = 16
_NW = _NC * _NS
_B = 16384
_D = 64
_BPW = _B // _NW
_SP = 128
_CH = 16
_NSTEP = _SP // _CH
_MARGIN = 1.0


def _sc_partials_body(ph, pr, pt, nh, nr, nt, ent, rel, out,
                      idx_h, idx_r, idx_t,
                      h_sp, r_sp, t_sp,
                      h_v, r_v, t_v, acc_v, sem, sem2):
    tid = lax.axis_index("s")
    wid = tid * _NC + lax.axis_index("c")
    base = wid * _BPW
    idxs = (idx_h, idx_r, idx_t)
    sps = (h_sp, r_sp, t_sp)
    vs = (h_v, r_v, t_v)
    tbls = (ent, rel, ent)

    def run_subphase(off, accs):
        def issue(k, _):
            for row in range(3):
                iv = idxs[row][pl.ds(off + k * _CH, _CH)]
                for i in range(_CH):
                    r = iv[i]
                    pltpu.async_copy(
                        tbls[row].at[pl.ds(r, 1)],
                        sps[row].at[tid, pl.ds(k * _CH + i, 1)], sem)
            return 0

        lax.fori_loop(0, _NSTEP, issue, 0)
        for row in range(3):
            pltpu.make_async_copy(
                tbls[row].at[pl.ds(0, _SP)],
                sps[row].at[tid], sem).wait()

        for row in range(3):
            pltpu.async_copy(sps[row].at[tid], vs[row], sem2)
        for row in range(3):
            pltpu.make_async_copy(sps[row].at[tid], vs[row], sem2).wait()

        def body(i, accs):
            new = []
            for j in range(_D // _L):
                sl = pl.ds(j * _L, _L)
                d = h_v[i, sl] + r_v[i, sl] - t_v[i, sl]
                new.append(accs[j] + jnp.abs(d))
            return tuple(new)

        return lax.fori_loop(0, _SP, body, accs)

    def run_phase(ih, ir, it):
        pltpu.sync_copy(ih.at[pl.ds(base, _BPW)], idx_h)
        pltpu.sync_copy(ir.at[pl.ds(base, _BPW)], idx_r)
        pltpu.sync_copy(it.at[pl.ds(base, _BPW)], idx_t)

        zero = jnp.zeros((_L,), jnp.float32)
        accs = (zero,) * (_D // _L)
        for sp in range(_BPW // _SP):
            accs = run_subphase(sp * _SP, accs)
        total = accs[0]
        for a in accs[1:]:
            total = total + a
        return total

    pos_sum = run_phase(ph, pr, pt)
    neg_sum = run_phase(nh, nr, nt)

    acc_v[...] = neg_sum - pos_sum
    pltpu.sync_copy(acc_v, out.at[wid])


_sc_partials = functools.partial(
    pl.kernel,
    out_type=jax.ShapeDtypeStruct((_NW, _L), jnp.float32),
    mesh=plsc.VectorSubcoreMesh(
        core_axis_name="c", subcore_axis_name="s",
        num_cores=_NC, num_subcores=_NS),
    scratch_types=[
        pltpu.VMEM((_BPW,), jnp.int32),
        pltpu.VMEM((_BPW,), jnp.int32),
        pltpu.VMEM((_BPW,), jnp.int32),
        pltpu.VMEM_SHARED((_NS, _SP, _D), jnp.float32),
        pltpu.VMEM_SHARED((_NS, _SP, _D), jnp.float32),
        pltpu.VMEM_SHARED((_NS, _SP, _D), jnp.float32),
        pltpu.VMEM((_SP, _D), jnp.float32),
        pltpu.VMEM((_SP, _D), jnp.float32),
        pltpu.VMEM((_SP, _D), jnp.float32),
        pltpu.VMEM((_L,), jnp.float32),
        pltpu.SemaphoreType.DMA,
        pltpu.SemaphoreType.DMA,
    ],
)(_sc_partials_body)


def _combine_body(parts_ref, out_ref):
    s = jnp.sum(parts_ref[...])
    out_ref[...] = jnp.maximum(s + _MARGIN, 0.0).reshape(1, 1)


_combine = pl.pallas_call(
    _combine_body,
    out_shape=jax.ShapeDtypeStruct((1, 1), jnp.float32),
)


@jax.jit
def kernel(pos_exmpl, neg_exmpl, entities_embeddings, relation_embeddings):
    ph, pr, pt = pos_exmpl[0], pos_exmpl[1], pos_exmpl[2]
    nh, nr, nt = neg_exmpl[0], neg_exmpl[1], neg_exmpl[2]
    parts = _sc_partials(ph, pr, pt, nh, nr, nt,
                         entities_embeddings, relation_embeddings)
    return _combine(parts)[0, 0]
